# baseline (speedup 1.0000x reference)
import functools

import jax
import jax.numpy as jnp
from jax import lax
from jax.experimental import pallas as pl
from jax.experimental.pallas import tpu as pltpu
from jax.experimental.pallas import tpu_sc as plsc

_NC, _NS = 2, 16
_NW = _NC * _NS


def _sc_mesh():
    return plsc.VectorSubcoreMesh(
        core_axis_name="c", subcore_axis_name="s",
        num_cores=_NC, num_subcores=_NS)


def _sc_gather2(table, idx_a, idx_b, chunk, name):
    M = idx_a.shape[0]
    D = table.shape[1]
    per_w = M // _NW
    n_ch = per_w // chunk
    assert per_w % chunk == 0 and M % _NW == 0

    @functools.partial(
        pl.kernel, mesh=_sc_mesh(),
        out_type=(jax.ShapeDtypeStruct((M, D), table.dtype),
                  jax.ShapeDtypeStruct((M, D), table.dtype)),
        scratch_types=[pltpu.VMEM((chunk,), jnp.int32),
                       pltpu.VMEM((chunk, D), table.dtype),
                       pltpu.VMEM((chunk,), jnp.int32),
                       pltpu.VMEM((chunk, D), table.dtype),
                       pltpu.SemaphoreType.DMA,
                       pltpu.SemaphoreType.DMA],
        name=name)
    def k(table_h, ia_h, ib_h, oa_h, ob_h, ia_v, ra_v, ib_v, rb_v, sa, sb):
        wid = lax.axis_index("s") * _NC + lax.axis_index("c")
        base = wid * per_w

        def body(j, carry):
            off = pl.multiple_of(base + j * chunk, 8)
            pltpu.sync_copy(ia_h.at[pl.ds(off, chunk)], ia_v)
            pltpu.sync_copy(ib_h.at[pl.ds(off, chunk)], ib_v)
            ca = pltpu.async_copy(table_h.at[ia_v], ra_v, sa)
            cb = pltpu.async_copy(table_h.at[ib_v], rb_v, sb)
            ca.wait()
            pltpu.sync_copy(ra_v, oa_h.at[pl.ds(off, chunk)])
            cb.wait()
            pltpu.sync_copy(rb_v, ob_h.at[pl.ds(off, chunk)])
            return carry

        lax.fori_loop(0, n_ch, body, 0)

    return k(table, idx_a, idx_b)


def _sc_gather1(table, idx, chunk, name):
    M = idx.shape[0]
    D = table.shape[1]
    per_w = M // _NW
    n_ch = per_w // chunk
    assert per_w % chunk == 0 and M % _NW == 0

    @functools.partial(
        pl.kernel, mesh=_sc_mesh(),
        out_type=jax.ShapeDtypeStruct((M, D), table.dtype),
        scratch_types=[pltpu.VMEM((chunk,), jnp.int32),
                       pltpu.VMEM((chunk, D), table.dtype),
                       pltpu.SemaphoreType.DMA],
        name=name)
    def k(table_h, idx_h, out_h, idx_v, rows_v, sem):
        wid = lax.axis_index("s") * _NC + lax.axis_index("c")
        base = wid * per_w

        def body(j, carry):
            off = pl.multiple_of(base + j * chunk, 8)
            pltpu.sync_copy(idx_h.at[pl.ds(off, chunk)], idx_v)
            pltpu.async_copy(table_h.at[idx_v], rows_v, sem).wait()
            pltpu.sync_copy(rows_v, out_h.at[pl.ds(off, chunk)])
            return carry

        lax.fori_loop(0, n_ch, body, 0)

    return k(table, idx)


def _zslice(nrows):
    for k in range(_NS, 0, -1):
        if nrows % k == 0 and (nrows // k) % 8 == 0:
            return k, nrows // k
    raise ValueError(nrows)


def _sc_segsum(vals, idx, nrows, chunk, zeros, name):
    M, D = vals.shape
    per_w = M // _NW
    n_ch = per_w // chunk
    n_zt, rows_t = _zslice(nrows)
    assert per_w % chunk == 0

    @functools.partial(
        pl.kernel, mesh=_sc_mesh(),
        out_type=jax.ShapeDtypeStruct((2 * nrows, D), vals.dtype),
        scratch_types=[pltpu.VMEM((chunk,), jnp.int32),
                       pltpu.VMEM((chunk, D), vals.dtype),
                       pltpu.VMEM_SHARED((nrows, D), vals.dtype),
                       pltpu.SemaphoreType.DMA],
        name=name)
    def k(vals_h, idx_h, zeros_h, out_h, idx_v, buf_v, acc_s, sem):
        c = lax.axis_index("c")
        s = lax.axis_index("s")
        wid = s * _NC + c
        z_off = pl.multiple_of(s * rows_t, 8)

        @pl.when(s < n_zt)
        def _():
            pltpu.sync_copy(zeros_h.at[pl.ds(z_off, rows_t)],
                            acc_s.at[pl.ds(z_off, rows_t)])

        plsc.subcore_barrier()
        base = wid * per_w

        def body(j, carry):
            off = pl.multiple_of(base + j * chunk, 8)
            pltpu.sync_copy(idx_h.at[pl.ds(off, chunk)], idx_v)
            pltpu.sync_copy(vals_h.at[pl.ds(off, chunk)], buf_v)
            pltpu.sync_copy(buf_v, acc_s.at[idx_v], add=True)
            return carry

        lax.fori_loop(0, n_ch, body, 0)
        plsc.subcore_barrier()

        @pl.when(s < n_zt)
        def _():
            pltpu.sync_copy(acc_s.at[pl.ds(z_off, rows_t)],
                            out_h.at[pl.ds(c * nrows + z_off, rows_t)])

    return k(vals, idx, zeros)


def _silu(x):
    return x * (1.0 / (1.0 + jnp.exp(-x)))


def _sigm(x):
    return 1.0 / (1.0 + jnp.exp(-x))


def _lrelu(x):
    return jnp.where(x >= 0, x, 0.01 * x)


def _mm(a, b):
    return jax.lax.dot(a, b, precision=jax.lax.Precision.DEFAULT)


def _row_spec(rb, d):
    return pl.BlockSpec((rb, d), lambda i: (i, 0))


def _full_spec(shape):
    return pl.BlockSpec(shape, lambda i: tuple(0 for _ in shape))


def kernel(H_0, Z, block_id, batch_id, edges, params):
    p = params
    N, Hd = H_0.shape
    E = edges.shape[1]
    NB = batch_id.shape[0]
    BS = 16
    f32 = jnp.float32

    src = edges[0].astype(jnp.int32)
    dst = edges[1].astype(jnp.int32)

    RBN = 1000
    BE = 3200
    GN = N // RBN
    GE = E // BE
    CH = 80

    pos = Z.reshape(N, 3)
    pos128 = jnp.pad(pos, ((0, 0), (0, Hd - 3)))

    def b_row(name):
        return p[name + '_b'].reshape(1, -1)

    def row1(v):
        return v.reshape(1, -1)

    def k_enc(h0_ref, we, be, wn, bn, xl_ref, xraw_ref):
        h0 = h0_ref[...]
        xl_ref[...] = _silu(_mm(h0, we[...]) + be[...])
        xraw_ref[...] = _silu(_mm(h0, wn[...]) + bn[...])

    xl, x_raw = pl.pallas_call(
        k_enc, grid=(GN,),
        in_specs=[_row_spec(RBN, Hd), _full_spec((Hd, Hd)), _full_spec((1, Hd)),
                  _full_spec((Hd, Hd)), _full_spec((1, Hd))],
        out_specs=[_row_spec(RBN, Hd), _row_spec(RBN, Hd)],
        out_shape=[jax.ShapeDtypeStruct((N, Hd), f32),
                   jax.ShapeDtypeStruct((N, Hd), f32)],
        name="tc_enc")(H_0, p['enc_lin_W'], b_row('enc_lin'),
                       p['lin_node_W'], b_row('lin_node'))
    x_psc = xl / jnp.maximum(jnp.linalg.norm(xl, axis=-1, keepdims=True),
                             1e-12) * 1.8

    ps128, pd128 = _sc_gather2(pos128, src, dst, CH, "sc_gather_pos")

    diff = ps128[:, :3] - pd128[:, :3]
    ew = jnp.sqrt(jnp.sum(diff ** 2, axis=1) + 1e-12)
    dist16 = jnp.broadcast_to(ew[:, None], (E, 16))

    loop = jnp.arange(N)
    s_c = jnp.concatenate([src, loop])
    d_c = jnp.concatenate([dst, loop])
    ww = jnp.concatenate([ew, jnp.ones((N,), f32)])
    deg = jax.ops.segment_sum(ww, d_c, num_segments=N)
    dinv = jnp.where(deg > 0, 1.0 / jnp.sqrt(deg), 0.0)
    norm = dinv[s_c] * ww * dinv[d_c]
    h = jax.ops.segment_sum(norm[:, None] * x_psc[s_c], d_c, num_segments=N)
    x_inter = 0.1 * x_psc + 0.9 * h

    def k_lin_lrelu(a_ref, b2_ref, w, b, t_ref):
        t_ref[...] = _lrelu(_mm(a_ref[...] + b2_ref[...], w[...]) + b[...])

    t_enc = pl.pallas_call(
        k_lin_lrelu, grid=(GN,),
        in_specs=[_row_spec(RBN, Hd), _row_spec(RBN, Hd),
                  _full_spec((Hd, Hd)), _full_spec((1, Hd))],
        out_specs=_row_spec(RBN, Hd),
        out_shape=jax.ShapeDtypeStruct((N, Hd), f32),
        name="tc_enc2")(x_inter, x_raw, p['mlp_enc_W'], b_row('mlp_enc'))

    def k_bn(t_ref, m, v, g, b, out_ref):
        out_ref[...] = (g[...] * (t_ref[...] - m[...])
                        / jnp.sqrt(v[...] + 1e-5) + b[...])

    def bn_apply(t, gname):
        dd = t.shape[1]
        return pl.pallas_call(
            k_bn, grid=(GN,),
            in_specs=[_row_spec(RBN, dd)] + [_full_spec((1, dd))] * 4,
            out_specs=_row_spec(RBN, dd),
            out_shape=jax.ShapeDtypeStruct(t.shape, f32),
            name="tc_bn_" + gname)(t, row1(t.mean(0)), row1(t.var(0)),
                                   row1(p[gname + '_g']), row1(p[gname + '_B']))

    x = bn_apply(t_enc, 'mlp_enc_bn')

    xs, xd = _sc_gather2(x, src, dst, CH, "sc_gather_x")
    eu_i = p['eu_inter_W']
    eu_a = p['eu_intra_W']

    def k_edge(xs_ref, xd_ref, d16_ref, mu_ref,
               wrbf_i, brbf_i, w1i, w2i, w3i, bi,
               wrbf_a, brbf_a, w1a, w2a, w3a, ba,
               mi_ref, ma_ref):
        xse = xs_ref[...]
        xde = xd_ref[...]
        d16 = d16_ref[...]
        mu = mu_ref[...]
        zz = (d16 - mu) / 0.375
        rbf = jnp.exp(-(zz * zz))
        ea = _sigm(_mm(rbf, wrbf_i[...]) + brbf_i[...])
        u = _silu(_mm(xse, w1i[...]) + _mm(xde, w2i[...]) + _mm(ea, w3i[...]) + bi[...])
        mi_ref[...] = jnp.maximum(xse + u, 0.0)
        ea = _sigm(_mm(rbf, wrbf_a[...]) + brbf_a[...])
        u = _silu(_mm(xse, w1a[...]) + _mm(xde, w2a[...]) + _mm(ea, w3a[...]) + ba[...])
        ma_ref[...] = jnp.maximum(xse + u, 0.0)

    msg_i, msg_a = pl.pallas_call(
        k_edge, grid=(GE,),
        in_specs=[_row_spec(BE, Hd), _row_spec(BE, Hd), _row_spec(BE, 16),
                  _full_spec((1, 16)),
                  _full_spec((16, Hd)), _full_spec((1, Hd)),
                  _full_spec((Hd, Hd)), _full_spec((Hd, Hd)),
                  _full_spec((Hd, Hd)), _full_spec((1, Hd)),
                  _full_spec((16, Hd)), _full_spec((1, Hd)),
                  _full_spec((Hd, Hd)), _full_spec((Hd, Hd)),
                  _full_spec((Hd, Hd)), _full_spec((1, Hd))],
        out_specs=[_row_spec(BE, Hd), _row_spec(BE, Hd)],
        out_shape=[jax.ShapeDtypeStruct((E, Hd), f32),
                   jax.ShapeDtypeStruct((E, Hd), f32)],
        name="tc_edge")(xs, xd, dist16,
                        jnp.linspace(0.0, 6.0, 16).reshape(1, 16),
                        p['ea_inter_W'], b_row('ea_inter'),
                        eu_i[:Hd], eu_i[Hd:2 * Hd], eu_i[2 * Hd:], b_row('eu_inter'),
                        p['ea_intra_W'], b_row('ea_intra'),
                        eu_a[:Hd], eu_a[Hd:2 * Hd], eu_a[2 * Hd:], b_row('eu_intra'))

    def seg(vals, idx, n):
        return jax.ops.segment_sum(vals, idx, num_segments=n)

    def k_add_lin_lrelu(x_ref, s_ref, w, b, t_ref):
        t_ref[...] = _lrelu(_mm(x_ref[...] + s_ref[...], w[...]) + b[...])

    def gine(segv, wname):
        t = pl.pallas_call(
            k_add_lin_lrelu, grid=(GN,),
            in_specs=[_row_spec(RBN, Hd), _row_spec(RBN, Hd),
                      _full_spec((Hd, Hd)), _full_spec((1, Hd))],
            out_specs=_row_spec(RBN, Hd),
            out_shape=jax.ShapeDtypeStruct((N, Hd), f32),
            name="tc_gine_" + wname)(x, segv, p[wname + '_W'], b_row(wname))
        return bn_apply(t, wname + '_bn')

    xi1 = gine(seg(msg_i, dst, N), 'gin1')
    xa1 = gine(seg(msg_a, dst, N), 'gin3')
    x_mask = gine(seg(x[src], dst, N), 'gin4')

    def k_dgnn(h_ref, a_ref, ws, bs, wn, bn, out_ref):
        out_ref[...] = _silu((_mm(h_ref[...], ws[...]) + bs[...])
                             + (_mm(a_ref[...], wn[...]) + bn[...]))

    def dgnn_tower(pref, tag):
        h = x
        for l in range(3):
            hs = _sc_gather1(h, src, CH, "sc_g_%s_%d" % (tag, l))
            agg = seg(hs, dst, N)
            h = pl.pallas_call(
                k_dgnn, grid=(GN,),
                in_specs=[_row_spec(RBN, Hd), _row_spec(RBN, Hd),
                          _full_spec((Hd, Hd)), _full_spec((1, Hd)),
                          _full_spec((Hd, Hd)), _full_spec((1, Hd))],
                out_specs=_row_spec(RBN, Hd),
                out_shape=jax.ShapeDtypeStruct((N, Hd), f32),
                name="tc_dgnn_%s_%d" % (tag, l))(
                    h, agg, p['%s_s%d_W' % (pref, l)], b_row('%s_s%d' % (pref, l)),
                    p['%s_n%d_W' % (pref, l)], b_row('%s_n%d' % (pref, l)))
        return h

    xi2 = dgnn_tower('dgnn1', 'd1')
    xa2 = dgnn_tower('dgnn3', 'd3')

    def k_lin2(a_ref, b2_ref, wa, wb, b, out_ref):
        out_ref[...] = _silu(_mm(a_ref[...], wa[...])
                             + _mm(b2_ref[...], wb[...]) + b[...])

    def lin2(x1, x2, wname, tag):
        w = p[wname + '_W']
        return pl.pallas_call(
            k_lin2, grid=(GN,),
            in_specs=[_row_spec(RBN, Hd), _row_spec(RBN, Hd),
                      _full_spec((Hd, Hd)), _full_spec((Hd, Hd)),
                      _full_spec((1, Hd))],
            out_specs=_row_spec(RBN, Hd),
            out_shape=jax.ShapeDtypeStruct((N, Hd), f32),
            name="tc_lin2_" + tag)(x1, x2, w[:Hd], w[Hd:], b_row(wname))

    x_inter2 = lin2(xi1, xi2, 'lin1', 'inter')
    x_intra = lin2(xa1, xa2, 'lin3', 'intra')

    H2 = 2 * Hd

    def k_fc0(xi_ref, xa_ref, xm_ref, w, b, t_ref):
        xc = xi_ref[...] + xa_ref[...] + xm_ref[...]
        t_ref[...] = _lrelu(_mm(xc, w[...]) + b[...])

    t0 = pl.pallas_call(
        k_fc0, grid=(GN,),
        in_specs=[_row_spec(RBN, Hd), _row_spec(RBN, Hd), _row_spec(RBN, Hd),
                  _full_spec((Hd, H2)), _full_spec((1, H2))],
        out_specs=_row_spec(RBN, H2),
        out_shape=jax.ShapeDtypeStruct((N, H2), f32),
        name="tc_fc0")(x_inter2, x_intra, x_mask, p['fc0_W'], b_row('fc0'))
    h0b = bn_apply(t0, 'fc0_bn')

    def k_fc(a_ref, w, b, t_ref):
        t_ref[...] = _lrelu(_mm(a_ref[...], w[...]) + b[...])

    def k_fc3(a_ref, w, b, t_ref):
        t_ref[...] = _mm(a_ref[...], w[...]) + b[...]

    def fc(a, wname, din, dout, lrelu=True):
        body = k_fc if lrelu else k_fc3
        return pl.pallas_call(
            body, grid=(GN,),
            in_specs=[_row_spec(RBN, din), _full_spec((din, dout)),
                      _full_spec((1, dout))],
            out_specs=_row_spec(RBN, dout),
            out_shape=jax.ShapeDtypeStruct((N, dout), f32),
            name="tc_" + wname)(a, p[wname + '_W'], b_row(wname))

    h1b = bn_apply(fc(h0b, 'fc1', H2, H2), 'fc1_bn')
    h2b = bn_apply(fc(h1b, 'fc2', H2, Hd), 'fc2_bn')
    H_upd = fc(h2b, 'fc3', Hd, Hd, lrelu=False)

    def norm2(v):
        return v / jnp.maximum(jnp.linalg.norm(v, axis=-1, keepdims=True), 1e-12)

    block_repr = norm2(seg(H_upd, block_id, NB))
    graph_repr = norm2(seg(block_repr, batch_id, BS))

    return (H_upd, block_repr, graph_repr, Z)
